# baseline (speedup 1.0000x reference)
import dataclasses
import functools

import jax
import jax.numpy as jnp
from jax import lax
from jax.experimental import pallas as pl
from jax.experimental.pallas import tpu as pltpu
from jax.experimental.pallas import tpu_sc as plsc

N = 10000
E = 160000
H = 16
C = 32
G = 100
HC = H * C

NC = 2
NS = 16
NW = NC * NS

NPAD = 10240
EPAD = 165888
EPT = EPAD // NW
EB = 48
NBLK = EPT // EB
NROWS_T = NPAD // NS
SENT = NPAD - 1

GPAD = 128
GSENT = GPAD - 1
RPT = NPAD // NW
RB = 64
GROWS_T = GPAD // NS


@functools.cache
def _mesh():
    return plsc.VectorSubcoreMesh(core_axis_name="c", subcore_axis_name="s")


@functools.cache
def _sc_params():
    cp = pltpu.CompilerParams()
    if "needs_layout_passes" in pltpu.CompilerParams.__dataclass_fields__:
        cp = dataclasses.replace(cp, needs_layout_passes=False)
    if "use_tc_tiling_on_sc" in pltpu.CompilerParams.__dataclass_fields__:
        cp = dataclasses.replace(cp, use_tc_tiling_on_sc=False)
    return cp


def _self_e(xl, xr, att_row):
    z = xl + xr
    z = jnp.maximum(z, z * 0.2)
    p = z * att_row
    blk = p.shape[0]
    return jnp.exp(p.reshape(blk, H, C).sum(-1))


def _dense2_body(x_ref, wl_ref, bl_ref, wr_ref, br_ref, att_ref,
                 xl_ref, xr_ref, es_ref):
    xb = x_ref[...]
    xl = jnp.dot(xb, wl_ref[...]) + bl_ref[...]
    xr = jnp.dot(xb, wr_ref[...]) + br_ref[...]
    xl_ref[...] = xl
    xr_ref[...] = xr
    es_ref[...] = _self_e(xl, xr, att_ref[...])


def _dense2(x, Wl, bl, Wr, br, att):
    K = x.shape[1]
    blk = 1024
    grid = NPAD // blk
    return pl.pallas_call(
        _dense2_body,
        grid=(grid,),
        in_specs=[
            pl.BlockSpec((blk, K), lambda i: (i, 0)),
            pl.BlockSpec((K, HC), lambda i: (0, 0)),
            pl.BlockSpec((1, HC), lambda i: (0, 0)),
            pl.BlockSpec((K, HC), lambda i: (0, 0)),
            pl.BlockSpec((1, HC), lambda i: (0, 0)),
            pl.BlockSpec((1, HC), lambda i: (0, 0)),
        ],
        out_specs=(
            pl.BlockSpec((blk, HC), lambda i: (i, 0)),
            pl.BlockSpec((blk, HC), lambda i: (i, 0)),
            pl.BlockSpec((blk, H), lambda i: (i, 0)),
        ),
        out_shape=(
            jax.ShapeDtypeStruct((NPAD, HC), jnp.float32),
            jax.ShapeDtypeStruct((NPAD, HC), jnp.float32),
            jax.ShapeDtypeStruct((NPAD, H), jnp.float32),
        ),
    )(x, Wl, bl.reshape(1, HC), Wr, br.reshape(1, HC), att.reshape(1, HC))


def _dense2_h_body(acc0_ref, acc1_ref, es_ref, den_ref, xlp_ref, bias_ref,
                   wl_ref, bl_ref, wr_ref, br_ref, att_ref,
                   xl_ref, xr_ref, es2_ref):
    alpha = es_ref[...] / den_ref[...]
    xlp = xlp_ref[...]
    msg = acc0_ref[...] + acc1_ref[...]
    for j in range(H):
        msg = msg + alpha[:, j:j + 1] * xlp[:, C * j:C * (j + 1)]
    hcur = msg * (1.0 / H) + bias_ref[...]
    hcur = jnp.maximum(hcur, hcur * 0.01)
    xl = jnp.dot(hcur, wl_ref[...]) + bl_ref[...]
    xr = jnp.dot(hcur, wr_ref[...]) + br_ref[...]
    xl_ref[...] = xl
    xr_ref[...] = xr
    es2_ref[...] = _self_e(xl, xr, att_ref[...])


def _dense2_h(acc_p, es, den, xl_prev, bias, Wl, bl, Wr, br, att):
    blk = 1024
    grid = NPAD // blk
    return pl.pallas_call(
        _dense2_h_body,
        grid=(grid,),
        in_specs=[
            pl.BlockSpec((blk, C), lambda i: (i, 0)),
            pl.BlockSpec((blk, C), lambda i: (i + NPAD // 1024, 0)),
            pl.BlockSpec((blk, H), lambda i: (i, 0)),
            pl.BlockSpec((blk, H), lambda i: (i, 0)),
            pl.BlockSpec((blk, HC), lambda i: (i, 0)),
            pl.BlockSpec((1, C), lambda i: (0, 0)),
            pl.BlockSpec((C, HC), lambda i: (0, 0)),
            pl.BlockSpec((1, HC), lambda i: (0, 0)),
            pl.BlockSpec((C, HC), lambda i: (0, 0)),
            pl.BlockSpec((1, HC), lambda i: (0, 0)),
            pl.BlockSpec((1, HC), lambda i: (0, 0)),
        ],
        out_specs=(
            pl.BlockSpec((blk, HC), lambda i: (i, 0)),
            pl.BlockSpec((blk, HC), lambda i: (i, 0)),
            pl.BlockSpec((blk, H), lambda i: (i, 0)),
        ),
        out_shape=(
            jax.ShapeDtypeStruct((NPAD, HC), jnp.float32),
            jax.ShapeDtypeStruct((NPAD, HC), jnp.float32),
            jax.ShapeDtypeStruct((NPAD, H), jnp.float32),
        ),
    )(acc_p, acc_p, es, den, xl_prev, bias.reshape(1, C),
      Wl, bl.reshape(1, HC), Wr, br.reshape(1, HC), att.reshape(1, HC))


def _pass1(xl, xr, idx2, att):
    @functools.partial(
        pl.kernel,
        out_type=(
            jax.ShapeDtypeStruct((EPAD, H), jnp.float32),
            jax.ShapeDtypeStruct((NC * NPAD, H), jnp.float32),
        ),
        mesh=_mesh(),
        compiler_params=_sc_params(),
        scratch_types=[
            pltpu.VMEM((2, EB), jnp.int32),
            pltpu.VMEM((2, EB), jnp.int32),
            pltpu.VMEM((EB, HC), jnp.float32),
            pltpu.VMEM((EB, HC), jnp.float32),
            pltpu.VMEM((EB, HC), jnp.float32),
            pltpu.VMEM((EB, HC), jnp.float32),
            pltpu.VMEM((EB, H), jnp.float32),
            pltpu.VMEM((NROWS_T, H), jnp.float32),
            pltpu.VMEM((H, C), jnp.float32),
            pltpu.VMEM_SHARED((NPAD, H), jnp.float32),
            pltpu.SemaphoreType.DMA,
            pltpu.SemaphoreType.DMA,
            pltpu.SemaphoreType.DMA,
            pltpu.SemaphoreType.DMA,
        ],
    )
    def k(xl_hbm, xr_hbm, idx_hbm, att_hbm, e_hbm, den_hbm,
          idxb0, idxb1, xlb0, xlb1, xrb0, xrb1, eb, zbuf, attb, den_sh,
          seml0, seml1, semr0, semr1):
        cid = lax.axis_index("c")
        sid = lax.axis_index("s")
        wid = sid * NC + cid

        zero = jnp.zeros((16,), jnp.float32)

        @pl.loop(0, NROWS_T)
        def _(i):
            zbuf[i, :] = zero

        pltpu.sync_copy(zbuf, den_sh.at[pl.ds(sid * NROWS_T, NROWS_T)])
        pltpu.sync_copy(att_hbm, attb)
        plsc.subcore_barrier()

        lane = lax.iota(jnp.int32, 16)
        att0 = [attb[j, pl.ds(0, 16)] for j in range(H)]
        att1 = [attb[j, pl.ds(16, 16)] for j in range(H)]
        base0 = wid * EPT

        slots = ((idxb0, xlb0, xrb0, seml0, semr0),
                 (idxb1, xlb1, xrb1, seml1, semr1))

        def issue(slot, b):
            idxb, xlb, xrb, seml, semr = slot
            base = base0 + b * EB
            pltpu.sync_copy(idx_hbm.at[:, pl.ds(base, EB)], idxb)
            pltpu.make_async_copy(xl_hbm.at[idxb.at[0]], xlb, seml).start()
            pltpu.make_async_copy(xr_hbm.at[idxb.at[1]], xrb, semr).start()

        def consume(slot, b):
            idxb, xlb, xrb, seml, semr = slot
            base = base0 + b * EB
            pltpu.make_async_copy(xl_hbm.at[idxb.at[0]], xlb, seml).wait()
            pltpu.make_async_copy(xr_hbm.at[idxb.at[1]], xrb, semr).wait()

            def edge_logits(ei):
                parts = []
                for j in range(H):
                    a0 = xlb[ei, pl.ds(C * j, 16)] + xrb[ei, pl.ds(C * j, 16)]
                    a1 = xlb[ei, pl.ds(C * j + 16, 16)] + xrb[ei, pl.ds(C * j + 16, 16)]
                    a0 = jnp.maximum(a0, a0 * 0.2)
                    a1 = jnp.maximum(a1, a1 * 0.2)
                    p = a0 * att0[j] + a1 * att1[j]
                    parts.append(jnp.where(lane == j, jnp.sum(p), 0.0))
                while len(parts) > 1:
                    parts = [parts[i] + parts[i + 1]
                             for i in range(0, len(parts), 2)]
                return parts[0]

            @pl.loop(0, EB, step=2)
            def _(ei):
                acc_a = edge_logits(ei)
                acc_b = edge_logits(ei + 1)
                eb[ei, :] = jnp.exp(acc_a)
                eb[ei + 1, :] = jnp.exp(acc_b)

            pltpu.sync_copy(eb, e_hbm.at[pl.ds(base, EB)])
            pltpu.sync_copy(eb, den_sh.at[idxb.at[1]], add=True)

        issue(slots[0], 0)

        @pl.loop(0, NBLK, step=2)
        def _(b):
            issue(slots[1], b + 1)
            consume(slots[0], b)

            @pl.when(b + 2 < NBLK)
            def _():
                issue(slots[0], b + 2)

            consume(slots[1], b + 1)

        plsc.subcore_barrier()
        r0 = sid * NROWS_T
        pltpu.sync_copy(den_sh.at[pl.ds(r0, NROWS_T)],
                        den_hbm.at[pl.ds(cid * NPAD + r0, NROWS_T)])

    return k(xl, xr, idx2, att)


def _bcast_lane(v, j):
    idx = jnp.full((16, 1), j, jnp.int32)
    dnums = lax.GatherDimensionNumbers(
        offset_dims=(), collapsed_slice_dims=(0,), start_index_map=(0,))
    return lax.gather(v, idx, dnums, (1,),
                      mode=lax.GatherScatterMode.PROMISE_IN_BOUNDS)


def _pass2(xl, e, den, idx2):
    @functools.partial(
        pl.kernel,
        out_type=jax.ShapeDtypeStruct((NC * NPAD, C), jnp.float32),
        mesh=_mesh(),
        compiler_params=_sc_params(),
        scratch_types=[
            pltpu.VMEM((2, EB), jnp.int32),
            pltpu.VMEM((2, EB), jnp.int32),
            pltpu.VMEM((EB, HC), jnp.float32),
            pltpu.VMEM((EB, HC), jnp.float32),
            pltpu.VMEM((EB, H), jnp.float32),
            pltpu.VMEM((EB, H), jnp.float32),
            pltpu.VMEM((EB, H), jnp.float32),
            pltpu.VMEM((EB, H), jnp.float32),
            pltpu.VMEM((EB, C), jnp.float32),
            pltpu.VMEM((NROWS_T, C), jnp.float32),
            pltpu.VMEM_SHARED((NPAD, C), jnp.float32),
            pltpu.SemaphoreType.DMA,
            pltpu.SemaphoreType.DMA,
            pltpu.SemaphoreType.DMA,
            pltpu.SemaphoreType.DMA,
            pltpu.SemaphoreType.DMA,
            pltpu.SemaphoreType.DMA,
        ],
    )
    def k(xl_hbm, e_hbm, den_hbm, idx_hbm, out_hbm,
          idxb0, idxb1, xlb0, xlb1, ebi0, ebi1, denb0, denb1, msgb, zbuf,
          out_sh, seml0, seml1, seme0, seme1, semd0, semd1):
        cid = lax.axis_index("c")
        sid = lax.axis_index("s")
        wid = sid * NC + cid

        zero = jnp.zeros((16,), jnp.float32)

        @pl.loop(0, NROWS_T)
        def _(i):
            zbuf[i, pl.ds(0, 16)] = zero
            zbuf[i, pl.ds(16, 16)] = zero

        pltpu.sync_copy(zbuf, out_sh.at[pl.ds(sid * NROWS_T, NROWS_T)])
        plsc.subcore_barrier()

        base0 = wid * EPT

        slots = ((idxb0, xlb0, ebi0, denb0, seml0, seme0, semd0),
                 (idxb1, xlb1, ebi1, denb1, seml1, seme1, semd1))

        def issue(slot, b):
            idxb, xlb, ebi, denb, seml, seme, semd = slot
            base = base0 + b * EB
            pltpu.sync_copy(idx_hbm.at[:, pl.ds(base, EB)], idxb)
            pltpu.make_async_copy(xl_hbm.at[idxb.at[0]], xlb, seml).start()
            pltpu.make_async_copy(den_hbm.at[idxb.at[1]], denb, semd).start()
            pltpu.make_async_copy(e_hbm.at[pl.ds(base, EB)], ebi, seme).start()

        def consume(slot, b):
            idxb, xlb, ebi, denb, seml, seme, semd = slot
            base = base0 + b * EB
            pltpu.make_async_copy(xl_hbm.at[idxb.at[0]], xlb, seml).wait()
            pltpu.make_async_copy(den_hbm.at[idxb.at[1]], denb, semd).wait()
            pltpu.make_async_copy(e_hbm.at[pl.ds(base, EB)], ebi, seme).wait()

            def edge_msg(ei):
                alpha = ebi[ei, :] / denb[ei, :]
                m0 = zero
                m1 = zero
                for j in range(H):
                    aj = _bcast_lane(alpha, j)
                    m0 = m0 + aj * xlb[ei, pl.ds(C * j, 16)]
                    m1 = m1 + aj * xlb[ei, pl.ds(C * j + 16, 16)]
                msgb[ei, pl.ds(0, 16)] = m0
                msgb[ei, pl.ds(16, 16)] = m1

            @pl.loop(0, EB, step=2)
            def _(ei):
                edge_msg(ei)
                edge_msg(ei + 1)

            pltpu.sync_copy(msgb, out_sh.at[idxb.at[1]], add=True)

        issue(slots[0], 0)

        @pl.loop(0, NBLK, step=2)
        def _(b):
            issue(slots[1], b + 1)
            consume(slots[0], b)

            @pl.when(b + 2 < NBLK)
            def _():
                issue(slots[0], b + 2)

            consume(slots[1], b + 1)

        plsc.subcore_barrier()
        r0 = sid * NROWS_T
        pltpu.sync_copy(out_sh.at[pl.ds(r0, NROWS_T)],
                        out_hbm.at[pl.ds(cid * NPAD + r0, NROWS_T)])

    return k(xl, e, den, idx2)


def _pool(acc_p, es, den, xl_prev, bias, batch):
    @functools.partial(
        pl.kernel,
        out_type=(
            jax.ShapeDtypeStruct((NC * GPAD, C), jnp.float32),
            jax.ShapeDtypeStruct((NC * GPAD, C), jnp.float32),
        ),
        mesh=_mesh(),
        compiler_params=_sc_params(),
        scratch_types=[
            pltpu.VMEM((1, RB), jnp.int32),
            pltpu.VMEM((RB, C), jnp.float32),
            pltpu.VMEM((RB, C), jnp.float32),
            pltpu.VMEM((RB, H), jnp.float32),
            pltpu.VMEM((RB, H), jnp.float32),
            pltpu.VMEM((RB, HC), jnp.float32),
            pltpu.VMEM((1, C), jnp.float32),
            pltpu.VMEM((RB, C), jnp.float32),
            pltpu.VMEM((RB, C), jnp.float32),
            pltpu.VMEM((GROWS_T, C), jnp.float32),
            pltpu.VMEM_SHARED((GPAD, C), jnp.float32),
            pltpu.VMEM_SHARED((GPAD, C), jnp.float32),
            pltpu.SemaphoreType.DMA,
        ],
    )
    def k(acc_hbm, es_hbm, den_hbm, xlp_hbm, bias_hbm, b_hbm,
          pooled_hbm, cnt_hbm,
          bidx, ab0, ab1, esb, denb, xlb, biasb, hb, ones, zbuf,
          pooled_sh, cnt_sh, sem1):
        cid = lax.axis_index("c")
        sid = lax.axis_index("s")
        wid = sid * NC + cid

        zero = jnp.zeros((16,), jnp.float32)
        one = jnp.ones((16,), jnp.float32)

        @pl.loop(0, GROWS_T)
        def _(i):
            zbuf[i, pl.ds(0, 16)] = zero
            zbuf[i, pl.ds(16, 16)] = zero

        @pl.loop(0, RB)
        def _(i):
            ones[i, pl.ds(0, 16)] = one
            ones[i, pl.ds(16, 16)] = one

        pltpu.sync_copy(zbuf, pooled_sh.at[pl.ds(sid * GROWS_T, GROWS_T)])
        pltpu.sync_copy(zbuf, cnt_sh.at[pl.ds(sid * GROWS_T, GROWS_T)])
        pltpu.sync_copy(bias_hbm, biasb)
        plsc.subcore_barrier()

        b0 = biasb[0, pl.ds(0, 16)]
        b1 = biasb[0, pl.ds(16, 16)]
        base0 = wid * RPT

        @pl.loop(0, RPT // RB)
        def _(b):
            base = base0 + b * RB
            pltpu.sync_copy(b_hbm.at[pl.ds(base, RB)], bidx.at[0])
            pltpu.sync_copy(acc_hbm.at[pl.ds(base, RB)], ab0)
            pltpu.sync_copy(acc_hbm.at[pl.ds(NPAD + base, RB)], ab1)
            pltpu.sync_copy(es_hbm.at[pl.ds(base, RB)], esb)
            pltpu.sync_copy(den_hbm.at[pl.ds(base, RB)], denb)
            pltpu.async_copy(xlp_hbm.at[pl.ds(base, RB)], xlb, sem1).wait()

            @pl.loop(0, RB)
            def _(ri):
                alpha = esb[ri, :] / denb[ri, :]
                m0 = ab0[ri, pl.ds(0, 16)] + ab1[ri, pl.ds(0, 16)]
                m1 = ab0[ri, pl.ds(16, 16)] + ab1[ri, pl.ds(16, 16)]
                for j in range(H):
                    aj = _bcast_lane(alpha, j)
                    m0 = m0 + aj * xlb[ri, pl.ds(C * j, 16)]
                    m1 = m1 + aj * xlb[ri, pl.ds(C * j + 16, 16)]
                h0 = m0 * (1.0 / H) + b0
                h1 = m1 * (1.0 / H) + b1
                hb[ri, pl.ds(0, 16)] = jnp.maximum(h0, h0 * 0.01)
                hb[ri, pl.ds(16, 16)] = jnp.maximum(h1, h1 * 0.01)

            pltpu.sync_copy(hb, pooled_sh.at[bidx.at[0]], add=True)
            pltpu.sync_copy(ones, cnt_sh.at[bidx.at[0]], add=True)

        plsc.subcore_barrier()
        r0 = sid * GROWS_T
        pltpu.sync_copy(pooled_sh.at[pl.ds(r0, GROWS_T)],
                        pooled_hbm.at[pl.ds(cid * GPAD + r0, GROWS_T)])
        pltpu.sync_copy(cnt_sh.at[pl.ds(r0, GROWS_T)],
                        cnt_hbm.at[pl.ds(cid * GPAD + r0, GROWS_T)])

    return k(acc_p, es, den, xl_prev, bias.reshape(1, C), batch)


def _head_body(p_ref, wc_ref, bc_ref, wlv_ref, blv_ref, lo_ref, lv_ref):
    p = p_ref[...]
    lo_ref[...] = jnp.dot(p, wc_ref[...]) + bc_ref[...]
    lv_ref[...] = jnp.dot(p, wlv_ref[...]) + blv_ref[...]


def _head(pooled, Wc, bc, Wlv, blv):
    nclass = bc.shape[0]
    return pl.pallas_call(
        _head_body,
        out_shape=(
            jax.ShapeDtypeStruct((G, nclass), jnp.float32),
            jax.ShapeDtypeStruct((G, 1), jnp.float32),
        ),
    )(pooled, Wc, bc.reshape(1, nclass), Wlv, blv.reshape(1, 1))


def kernel(x, edge_index, batch, Wl1, bl1, Wr1, br1, att1, bias1,
           Wl2, bl2, Wr2, br2, att2, bias2, Wc, bc, Wlv, blv):
    idx2 = jnp.pad(edge_index, ((0, 0), (0, EPAD - E)),
                   constant_values=SENT)

    xp = jnp.zeros((NPAD, x.shape[1]), x.dtype).at[:N].set(x)

    xl1, xr1, es1 = _dense2(xp, Wl1, bl1, Wr1, br1, att1)
    e1, den1p = _pass1(xl1, xr1, idx2, att1)
    den1 = den1p[:NPAD] + den1p[NPAD:] + es1 + 1e-16
    acc1 = _pass2(xl1, e1, den1, idx2)

    xl2, xr2, es2 = _dense2_h(acc1, es1, den1, xl1, bias1,
                              Wl2, bl2, Wr2, br2, att2)
    e2, den2p = _pass1(xl2, xr2, idx2, att2)
    den2 = den2p[:NPAD] + den2p[NPAD:] + es2 + 1e-16
    acc2 = _pass2(xl2, e2, den2, idx2)

    batchp = jnp.concatenate(
        [batch, jnp.full((NPAD - N,), GSENT, batch.dtype)])
    pooled_p, cnt_p = _pool(acc2, es2, den2, xl2, bias2, batchp)
    pooled = (pooled_p[:GPAD] + pooled_p[GPAD:])[:G]
    cnt = (cnt_p[:GPAD] + cnt_p[GPAD:])[:G]
    pooled = pooled / jnp.clip(cnt, 1.0)
    return _head(pooled, Wc, bc, Wlv, blv)


# speedup vs baseline: 1.5210x; 1.5210x over previous
import dataclasses
import functools

import jax
import jax.numpy as jnp
from jax import lax
from jax.experimental import pallas as pl
from jax.experimental.pallas import tpu as pltpu
from jax.experimental.pallas import tpu_sc as plsc

N = 10000
E = 160000
H = 16
C = 32
G = 100
HC = H * C

NC = 2
NS = 16
NW = NC * NS

NPAD = 10240
EPAD = 172032
EPT = EPAD // NW
EB1 = 48
NBLK1 = EPT // EB1
EB2 = 64
NBLK2 = EPT // EB2
NROWS_T = NPAD // NS
SENT = NPAD - 1

GPAD = 128
GSENT = GPAD - 1
RPT = NPAD // NW
GROWS_T = GPAD // NS


@functools.cache
def _mesh():
    return plsc.VectorSubcoreMesh(core_axis_name="c", subcore_axis_name="s")


@functools.cache
def _sc_params():
    cp = pltpu.CompilerParams()
    if "needs_layout_passes" in pltpu.CompilerParams.__dataclass_fields__:
        cp = dataclasses.replace(cp, needs_layout_passes=False)
    if "use_tc_tiling_on_sc" in pltpu.CompilerParams.__dataclass_fields__:
        cp = dataclasses.replace(cp, use_tc_tiling_on_sc=False)
    return cp


def _dense2_body(x_ref, wl_ref, bl_ref, wr_ref, br_ref, xl_ref, xr_ref):
    xb = x_ref[...]
    xl_ref[...] = jnp.dot(xb, wl_ref[...]) + bl_ref[...]
    xr_ref[...] = jnp.dot(xb, wr_ref[...]) + br_ref[...]


def _dense2(x, Wl, bl, Wr, br):
    K = x.shape[1]
    blk = 1024
    grid = NPAD // blk
    return pl.pallas_call(
        _dense2_body,
        grid=(grid,),
        in_specs=[
            pl.BlockSpec((blk, K), lambda i: (i, 0)),
            pl.BlockSpec((K, HC), lambda i: (0, 0)),
            pl.BlockSpec((1, HC), lambda i: (0, 0)),
            pl.BlockSpec((K, HC), lambda i: (0, 0)),
            pl.BlockSpec((1, HC), lambda i: (0, 0)),
        ],
        out_specs=(
            pl.BlockSpec((blk, HC), lambda i: (i, 0)),
            pl.BlockSpec((blk, HC), lambda i: (i, 0)),
        ),
        out_shape=(
            jax.ShapeDtypeStruct((NPAD, HC), jnp.float32),
            jax.ShapeDtypeStruct((NPAD, HC), jnp.float32),
        ),
    )(x, Wl, bl.reshape(1, HC), Wr, br.reshape(1, HC))


def _pass1(xl, xr, idx2, att):
    @functools.partial(
        pl.kernel,
        out_type=(
            jax.ShapeDtypeStruct((EPAD, H), jnp.float32),
            jax.ShapeDtypeStruct((NC * NPAD, H), jnp.float32),
        ),
        mesh=_mesh(),
        compiler_params=_sc_params(),
        scratch_types=[
            pltpu.VMEM((NBLK1, 2, EB1), jnp.int32),
            pltpu.VMEM((EB1, HC), jnp.float32),
            pltpu.VMEM((EB1, HC), jnp.float32),
            pltpu.VMEM((EB1, HC), jnp.float32),
            pltpu.VMEM((EB1, HC), jnp.float32),
            pltpu.VMEM((EB1, H), jnp.float32),
            pltpu.VMEM((64, H), jnp.float32),
            pltpu.VMEM((H, C), jnp.float32),
            pltpu.VMEM_SHARED((NPAD, H), jnp.float32),
            pltpu.SemaphoreType.DMA,
            pltpu.SemaphoreType.DMA,
            pltpu.SemaphoreType.DMA,
            pltpu.SemaphoreType.DMA,
        ],
    )
    def k(xl_hbm, xr_hbm, idx_hbm, att_hbm, e_hbm, den_hbm,
          idxall, xlb0, xlb1, xrb0, xrb1, eb, zbuf, attb, den_sh,
          seml0, seml1, semr0, semr1):
        cid = lax.axis_index("c")
        sid = lax.axis_index("s")
        wid = sid * NC + cid

        zero = jnp.zeros((16,), jnp.float32)

        @pl.loop(0, 64)
        def _(i):
            zbuf[i, :] = zero

        @pl.loop(0, NROWS_T // 64)
        def _(i):
            pltpu.sync_copy(zbuf, den_sh.at[pl.ds(sid * NROWS_T + i * 64, 64)])

        pltpu.sync_copy(att_hbm, attb)
        pltpu.sync_copy(idx_hbm.at[wid], idxall)
        plsc.subcore_barrier()

        lane = lax.iota(jnp.int32, 16)
        att0 = [attb[j, pl.ds(0, 16)] for j in range(H)]
        att1 = [attb[j, pl.ds(16, 16)] for j in range(H)]
        base0 = wid * EPT

        slots = ((xlb0, xrb0, seml0, semr0),
                 (xlb1, xrb1, seml1, semr1))

        def issue(slot, b):
            xlb, xrb, seml, semr = slot
            pltpu.make_async_copy(xl_hbm.at[idxall.at[b, 0]], xlb, seml).start()
            pltpu.make_async_copy(xr_hbm.at[idxall.at[b, 1]], xrb, semr).start()

        def consume(slot, b):
            xlb, xrb, seml, semr = slot
            base = base0 + b * EB1
            pltpu.make_async_copy(xl_hbm.at[idxall.at[b, 0]], xlb, seml).wait()
            pltpu.make_async_copy(xr_hbm.at[idxall.at[b, 1]], xrb, semr).wait()

            def edge_logits(ei):
                parts = []
                for j in range(H):
                    a0 = xlb[ei, pl.ds(C * j, 16)] + xrb[ei, pl.ds(C * j, 16)]
                    a1 = xlb[ei, pl.ds(C * j + 16, 16)] + xrb[ei, pl.ds(C * j + 16, 16)]
                    a0 = jnp.maximum(a0, a0 * 0.2)
                    a1 = jnp.maximum(a1, a1 * 0.2)
                    p = a0 * att0[j] + a1 * att1[j]
                    parts.append(jnp.where(lane == j, jnp.sum(p), 0.0))
                while len(parts) > 1:
                    parts = [parts[i] + parts[i + 1]
                             for i in range(0, len(parts), 2)]
                return parts[0]

            @pl.loop(0, EB1, step=2)
            def _(ei):
                acc_a = edge_logits(ei)
                acc_b = edge_logits(ei + 1)
                eb[ei, :] = jnp.exp(acc_a)
                eb[ei + 1, :] = jnp.exp(acc_b)

            pltpu.sync_copy(eb, e_hbm.at[pl.ds(base, EB1)])
            pltpu.sync_copy(eb, den_sh.at[idxall.at[b, 1]], add=True)

        issue(slots[0], 0)

        @pl.loop(0, NBLK1, step=2)
        def _(b):
            issue(slots[1], b + 1)
            consume(slots[0], b)

            @pl.when(b + 2 < NBLK1)
            def _():
                issue(slots[0], b + 2)

            consume(slots[1], b + 1)

        plsc.subcore_barrier()
        r0 = sid * NROWS_T
        pltpu.sync_copy(den_sh.at[pl.ds(r0, NROWS_T)],
                        den_hbm.at[pl.ds(cid * NPAD + r0, NROWS_T)])

    return k(xl, xr, idx2, att)


def _bcast_lane(v, j):
    idx = jnp.full((16, 1), j, jnp.int32)
    dnums = lax.GatherDimensionNumbers(
        offset_dims=(), collapsed_slice_dims=(0,), start_index_map=(0,))
    return lax.gather(v, idx, dnums, (1,),
                      mode=lax.GatherScatterMode.PROMISE_IN_BOUNDS)


def _pass2(xl, e, den, idx2):
    @functools.partial(
        pl.kernel,
        out_type=jax.ShapeDtypeStruct((NC * NPAD, C), jnp.float32),
        mesh=_mesh(),
        compiler_params=_sc_params(),
        scratch_types=[
            pltpu.VMEM((NBLK2, 2, EB2), jnp.int32),
            pltpu.VMEM((EB2, HC), jnp.float32),
            pltpu.VMEM((EB2, HC), jnp.float32),
            pltpu.VMEM((EB2, H), jnp.float32),
            pltpu.VMEM((EB2, H), jnp.float32),
            pltpu.VMEM((EB2, H), jnp.float32),
            pltpu.VMEM((EB2, H), jnp.float32),
            pltpu.VMEM((EB2, C), jnp.float32),
            pltpu.VMEM((64, C), jnp.float32),
            pltpu.VMEM_SHARED((NPAD, C), jnp.float32),
            pltpu.SemaphoreType.DMA,
            pltpu.SemaphoreType.DMA,
            pltpu.SemaphoreType.DMA,
            pltpu.SemaphoreType.DMA,
            pltpu.SemaphoreType.DMA,
            pltpu.SemaphoreType.DMA,
        ],
    )
    def k(xl_hbm, e_hbm, den_hbm, idx_hbm, out_hbm,
          idxall, xlb0, xlb1, ebi0, ebi1, denb0, denb1, msgb, zbuf,
          out_sh, seml0, seml1, seme0, seme1, semd0, semd1):
        cid = lax.axis_index("c")
        sid = lax.axis_index("s")
        wid = sid * NC + cid

        zero = jnp.zeros((16,), jnp.float32)

        @pl.loop(0, 64)
        def _(i):
            zbuf[i, pl.ds(0, 16)] = zero
            zbuf[i, pl.ds(16, 16)] = zero

        @pl.loop(0, NROWS_T // 64)
        def _(i):
            pltpu.sync_copy(zbuf, out_sh.at[pl.ds(sid * NROWS_T + i * 64, 64)])

        pltpu.sync_copy(idx_hbm.at[wid], idxall)
        plsc.subcore_barrier()

        base0 = wid * EPT

        slots = ((xlb0, ebi0, denb0, seml0, seme0, semd0),
                 (xlb1, ebi1, denb1, seml1, seme1, semd1))

        def issue(slot, b):
            xlb, ebi, denb, seml, seme, semd = slot
            base = base0 + b * EB2
            pltpu.make_async_copy(xl_hbm.at[idxall.at[b, 0]], xlb, seml).start()
            pltpu.make_async_copy(den_hbm.at[idxall.at[b, 1]], denb, semd).start()
            pltpu.make_async_copy(e_hbm.at[pl.ds(base, EB2)], ebi, seme).start()

        def consume(slot, b):
            xlb, ebi, denb, seml, seme, semd = slot
            base = base0 + b * EB2
            pltpu.make_async_copy(xl_hbm.at[idxall.at[b, 0]], xlb, seml).wait()
            pltpu.make_async_copy(den_hbm.at[idxall.at[b, 1]], denb, semd).wait()
            pltpu.make_async_copy(e_hbm.at[pl.ds(base, EB2)], ebi, seme).wait()

            def edge_msg(ei):
                alpha = ebi[ei, :] / denb[ei, :]
                m0 = zero
                m1 = zero
                for j in range(H):
                    aj = _bcast_lane(alpha, j)
                    m0 = m0 + aj * xlb[ei, pl.ds(C * j, 16)]
                    m1 = m1 + aj * xlb[ei, pl.ds(C * j + 16, 16)]
                msgb[ei, pl.ds(0, 16)] = m0
                msgb[ei, pl.ds(16, 16)] = m1

            @pl.loop(0, EB2, step=2)
            def _(ei):
                edge_msg(ei)
                edge_msg(ei + 1)

            pltpu.sync_copy(msgb, out_sh.at[idxall.at[b, 1]], add=True)

        issue(slots[0], 0)

        @pl.loop(0, NBLK2, step=2)
        def _(b):
            issue(slots[1], b + 1)
            consume(slots[0], b)

            @pl.when(b + 2 < NBLK2)
            def _():
                issue(slots[0], b + 2)

            consume(slots[1], b + 1)

        plsc.subcore_barrier()
        r0 = sid * NROWS_T
        pltpu.sync_copy(out_sh.at[pl.ds(r0, NROWS_T)],
                        out_hbm.at[pl.ds(cid * NPAD + r0, NROWS_T)])

    return k(xl, e, den, idx2)


def _pool(h, batch):
    @functools.partial(
        pl.kernel,
        out_type=(
            jax.ShapeDtypeStruct((NC * GPAD, C), jnp.float32),
            jax.ShapeDtypeStruct((NC * GPAD, C), jnp.float32),
        ),
        mesh=_mesh(),
        compiler_params=_sc_params(),
        scratch_types=[
            pltpu.VMEM((1, EB2), jnp.int32),
            pltpu.VMEM((EB2, C), jnp.float32),
            pltpu.VMEM((EB2, C), jnp.float32),
            pltpu.VMEM((GROWS_T, C), jnp.float32),
            pltpu.VMEM_SHARED((GPAD, C), jnp.float32),
            pltpu.VMEM_SHARED((GPAD, C), jnp.float32),
            pltpu.SemaphoreType.DMA,
        ],
    )
    def k(h_hbm, b_hbm, pooled_hbm, cnt_hbm,
          bidx, hb, ones, zbuf, pooled_sh, cnt_sh, sem1):
        cid = lax.axis_index("c")
        sid = lax.axis_index("s")
        wid = sid * NC + cid

        zero = jnp.zeros((16,), jnp.float32)
        one = jnp.ones((16,), jnp.float32)

        @pl.loop(0, GROWS_T)
        def _(i):
            zbuf[i, pl.ds(0, 16)] = zero
            zbuf[i, pl.ds(16, 16)] = zero

        @pl.loop(0, EB2)
        def _(i):
            ones[i, pl.ds(0, 16)] = one
            ones[i, pl.ds(16, 16)] = one

        pltpu.sync_copy(zbuf, pooled_sh.at[pl.ds(sid * GROWS_T, GROWS_T)])
        pltpu.sync_copy(zbuf, cnt_sh.at[pl.ds(sid * GROWS_T, GROWS_T)])
        plsc.subcore_barrier()

        base0 = wid * RPT

        @pl.loop(0, RPT // EB2)
        def _(b):
            base = base0 + b * EB2
            pltpu.sync_copy(b_hbm.at[pl.ds(base, EB2)], bidx.at[0])
            pltpu.async_copy(h_hbm.at[pl.ds(base, EB2)], hb, sem1).wait()
            pltpu.sync_copy(hb, pooled_sh.at[bidx.at[0]], add=True)
            pltpu.sync_copy(ones, cnt_sh.at[bidx.at[0]], add=True)

        plsc.subcore_barrier()
        r0 = sid * GROWS_T
        pltpu.sync_copy(pooled_sh.at[pl.ds(r0, GROWS_T)],
                        pooled_hbm.at[pl.ds(cid * GPAD + r0, GROWS_T)])
        pltpu.sync_copy(cnt_sh.at[pl.ds(r0, GROWS_T)],
                        cnt_hbm.at[pl.ds(cid * GPAD + r0, GROWS_T)])

    return k(h, batch)


def _head_body(p_ref, wc_ref, bc_ref, wlv_ref, blv_ref, lo_ref, lv_ref):
    p = p_ref[...]
    lo_ref[...] = jnp.dot(p, wc_ref[...]) + bc_ref[...]
    lv_ref[...] = jnp.dot(p, wlv_ref[...]) + blv_ref[...]


def _head(pooled, Wc, bc, Wlv, blv):
    nclass = bc.shape[0]
    return pl.pallas_call(
        _head_body,
        out_shape=(
            jax.ShapeDtypeStruct((G, nclass), jnp.float32),
            jax.ShapeDtypeStruct((G, 1), jnp.float32),
        ),
    )(pooled, Wc, bc.reshape(1, nclass), Wlv, blv.reshape(1, 1))


def _gat_layer(x, idxa, idxb, Wl, bl, Wr, br, att, bias):
    xl, xr = _dense2(x, Wl, bl, Wr, br)
    e, den_p = _pass1(xl, xr, idxa, att)
    den = den_p[:NPAD] + den_p[NPAD:] + 1e-16
    acc = _pass2(xl, e, den, idxb)
    return (acc[:NPAD] + acc[NPAD:]) * (1.0 / H) + bias


def kernel(x, edge_index, batch, Wl1, bl1, Wr1, br1, att1, bias1,
           Wl2, bl2, Wr2, br2, att2, bias2, Wc, bc, Wlv, blv):
    loop = jnp.arange(N, dtype=edge_index.dtype)
    fill = jnp.full((EPAD - E - N,), SENT, edge_index.dtype)
    src = jnp.concatenate([edge_index[0], loop, fill])
    dst = jnp.concatenate([edge_index[1], loop, fill])
    idx2 = jnp.stack([src, dst])
    idxa = idx2.reshape(2, NW, NBLK1, EB1).transpose(1, 2, 0, 3)
    idxb = idx2.reshape(2, NW, NBLK2, EB2).transpose(1, 2, 0, 3)

    xp = jnp.zeros((NPAD, x.shape[1]), x.dtype).at[:N].set(x)
    h = _gat_layer(xp, idxa, idxb, Wl1, bl1, Wr1, br1, att1, bias1)
    h = jax.nn.leaky_relu(h, 0.01)
    h = _gat_layer(h, idxa, idxb, Wl2, bl2, Wr2, br2, att2, bias2)
    h = jax.nn.leaky_relu(h, 0.01)

    batchp = jnp.concatenate(
        [batch, jnp.full((NPAD - N,), GSENT, batch.dtype)])
    pooled_p, cnt_p = _pool(h, batchp)
    pooled = (pooled_p[:GPAD] + pooled_p[GPAD:])[:G]
    cnt = (cnt_p[:GPAD] + cnt_p[GPAD:])[:G]
    pooled = pooled / jnp.clip(cnt, 1.0)
    return _head(pooled, Wc, bc, Wlv, blv)
